# baseline (speedup 1.0000x reference)
import jax
import jax.numpy as jnp
from jax import lax
from jax.experimental import pallas as pl
from jax.experimental.pallas import tpu as pltpu
from jax.experimental.pallas import tpu_sc as plsc

N = 50000
E = 800000
B = 128
NC = 2
NS = 16

K = 125
NI = 16
P = 4

WB = 3128
WB_LAST = N - 15 * WB

GR = 250
NGRP = N // GR

R = 5000
GRID = N // R


def _zero_acc(zeros_hbm, acc, s):
    @pl.when(s < NS - 1)
    def _():
        pltpu.sync_copy(zeros_hbm, acc.at[pl.ds(s * WB, WB)])

    @pl.when(s == NS - 1)
    def _():
        pltpu.sync_copy(zeros_hbm.at[pl.ds(0, WB_LAST)],
                        acc.at[pl.ds((NS - 1) * WB, WB_LAST)])


def _edge_loop(m_hbm, e3_hbm, acc, sidx, didx, rows, sems, s):
    n_outer = E // (NS * NI * K)
    base_blk = s * n_outer

    def outer(i, carry):
        r0 = (base_blk + i) * NI
        pltpu.sync_copy(e3_hbm.at[0].at[pl.ds(r0, NI)], sidx)
        pltpu.sync_copy(e3_hbm.at[1].at[pl.ds(r0, NI)], didx)
        cps = [None] * NI
        for j in range(P):
            cps[j] = pltpu.async_copy(m_hbm.at[sidx.at[j]], rows.at[j % P],
                                      sems.at[j % P])
        for j in range(NI):
            cps[j].wait()
            pltpu.sync_copy(rows.at[j % P], acc.at[didx.at[j]], add=True)
            if j + P < NI:
                cps[j + P] = pltpu.async_copy(m_hbm.at[sidx.at[j + P]],
                                              rows.at[j % P], sems.at[j % P])
        return carry

    lax.fori_loop(0, n_outer, outer, 0)


def _finish_group(g, acc, hp, out, gid3, g_acc, mode):
    def scoped(vbuf, vbuf2, gbuf):
        r0 = g * GR
        pltpu.sync_copy(acc.at[pl.ds(r0, GR)], vbuf)
        if mode >= 2:
            pltpu.sync_copy(hp.at[pl.ds(r0, GR)], vbuf2)

        def row(r, carry):
            for off in (0, 16):
                v = vbuf[r, pl.ds(off, 16)]
                if mode >= 2:
                    v = v + vbuf2[r, pl.ds(off, 16)]
                vbuf[r, pl.ds(off, 16)] = jnp.maximum(v, 0.0)
            return carry

        lax.fori_loop(0, GR, row, 0)
        if mode == 3:
            pltpu.sync_copy(gid3.at[g], gbuf)
            for j in range(2):
                pltpu.sync_copy(vbuf.at[pl.ds(j * K, K)],
                                g_acc.at[gbuf.at[j]], add=True)
        else:
            pltpu.sync_copy(vbuf, out.at[pl.ds(r0, GR)])

    pl.run_scoped(scoped,
                  pltpu.VMEM((GR, 32), jnp.float32),
                  pltpu.VMEM((GR, 32), jnp.float32),
                  pltpu.VMEM((2, K), jnp.int32))


def _finish(acc, hp, out, gid3, g_acc, s, mode):
    for k in range(12):
        _finish_group(s + NS * k, acc, hp, out, gid3, g_acc, mode)

    @pl.when(s < NGRP - 12 * NS)
    def _():
        _finish_group(12 * NS + s, acc, hp, out, gid3, g_acc, mode)


def _edges_both_cores(m0, m1, e3, acc, sidx, didx, s, c):
    def scoped(rows, sems):
        @pl.when(c == 0)
        def _():
            _edge_loop(m0, e3, acc, sidx, didx, rows, sems, s)

        @pl.when(c == 1)
        def _():
            _edge_loop(m1, e3, acc, sidx, didx, rows, sems, s)

    pl.run_scoped(scoped,
                  pltpu.VMEM((P, K, 32), jnp.float32),
                  pltpu.SemaphoreType.DMA((P,)))


def _layer1_body(m0, m1, e3, zeros, out0, out1, acc, sidx, didx):
    c = lax.axis_index("c")
    s = lax.axis_index("s")
    _zero_acc(zeros, acc, s)
    plsc.subcore_barrier()
    _edges_both_cores(m0, m1, e3, acc, sidx, didx, s, c)
    plsc.subcore_barrier()

    @pl.when(c == 0)
    def _():
        _finish(acc, None, out0, None, None, s, mode=1)

    @pl.when(c == 1)
    def _():
        _finish(acc, None, out1, None, None, s, mode=1)


def _layer2_body(m0, m1, e3, zeros, hp0, hp1, out0, out1, acc, sidx, didx):
    c = lax.axis_index("c")
    s = lax.axis_index("s")
    _zero_acc(zeros, acc, s)
    plsc.subcore_barrier()
    _edges_both_cores(m0, m1, e3, acc, sidx, didx, s, c)
    plsc.subcore_barrier()

    @pl.when(c == 0)
    def _():
        _finish(acc, hp0, out0, None, None, s, mode=2)

    @pl.when(c == 1)
    def _():
        _finish(acc, hp1, out1, None, None, s, mode=2)


def _layer3_body(m0, m1, e3, zeros, hp0, hp1, gid3, gout0, gout1,
                 acc, g_acc, sidx, didx):
    c = lax.axis_index("c")
    s = lax.axis_index("s")
    _zero_acc(zeros, acc, s)

    @pl.when(s == 0)
    def _():
        pltpu.sync_copy(zeros.at[pl.ds(0, B)], g_acc)

    plsc.subcore_barrier()
    _edges_both_cores(m0, m1, e3, acc, sidx, didx, s, c)
    plsc.subcore_barrier()

    @pl.when(c == 0)
    def _():
        _finish(acc, hp0, None, gid3, g_acc, s, mode=3)

    @pl.when(c == 1)
    def _():
        _finish(acc, hp1, None, gid3, g_acc, s, mode=3)

    plsc.subcore_barrier()

    @pl.when((s == 0) & (c == 0))
    def _():
        pltpu.sync_copy(g_acc, gout0)

    @pl.when((s == 0) & (c == 1))
    def _():
        pltpu.sync_copy(g_acc, gout1)


def _sc_mesh():
    return plsc.VectorSubcoreMesh(core_axis_name="c", subcore_axis_name="s",
                                  num_cores=NC, num_subcores=NS)


_H32 = jax.ShapeDtypeStruct((N, 32), jnp.float32)
_G32 = jax.ShapeDtypeStruct((B, 32), jnp.float32)


def _sc_layer1(m0, m1, e3, zeros):
    f = pl.kernel(
        _layer1_body, out_type=(_H32, _H32), mesh=_sc_mesh(),
        compiler_params=pltpu.CompilerParams(use_tc_tiling_on_sc=False),
        scratch_types=[
            pltpu.VMEM_SHARED((N, 32), jnp.float32),
            pltpu.VMEM((NI, K), jnp.int32),
            pltpu.VMEM((NI, K), jnp.int32),
        ],
    )
    return f(m0, m1, e3, zeros)


def _sc_layer2(m0, m1, e3, zeros, hp0, hp1):
    f = pl.kernel(
        _layer2_body, out_type=(_H32, _H32), mesh=_sc_mesh(),
        compiler_params=pltpu.CompilerParams(use_tc_tiling_on_sc=False),
        scratch_types=[
            pltpu.VMEM_SHARED((N, 32), jnp.float32),
            pltpu.VMEM((NI, K), jnp.int32),
            pltpu.VMEM((NI, K), jnp.int32),
        ],
    )
    return f(m0, m1, e3, zeros, hp0, hp1)


def _sc_layer3(m0, m1, e3, zeros, hp0, hp1, gid3):
    f = pl.kernel(
        _layer3_body, out_type=(_G32, _G32), mesh=_sc_mesh(),
        compiler_params=pltpu.CompilerParams(use_tc_tiling_on_sc=False),
        scratch_types=[
            pltpu.VMEM_SHARED((N, 32), jnp.float32),
            pltpu.VMEM_SHARED((B, 32), jnp.float32),
            pltpu.VMEM((NI, K), jnp.int32),
            pltpu.VMEM((NI, K), jnp.int32),
        ],
    )
    return f(m0, m1, e3, zeros, hp0, hp1, gid3)


def _emb_body(x_ref, we_ref, w0_ref, o0_ref, o1_ref):
    h0 = jnp.dot(x_ref[...], we_ref[...], preferred_element_type=jnp.float32)
    m0 = jnp.dot(h0, w0_ref[...], preferred_element_type=jnp.float32)
    o0_ref[...] = m0[:, :32]
    o1_ref[...] = m0[:, 32:]


def _tc_emb(x, w_emb, w_gcn0):
    return pl.pallas_call(
        _emb_body,
        grid=(GRID,),
        in_specs=[pl.BlockSpec((R, 27), lambda i: (i, 0)),
                  pl.BlockSpec((27, 8), lambda i: (0, 0)),
                  pl.BlockSpec((8, 64), lambda i: (0, 0))],
        out_specs=(pl.BlockSpec((R, 32), lambda i: (i, 0)),
                   pl.BlockSpec((R, 32), lambda i: (i, 0))),
        out_shape=(_H32, _H32),
    )(x, w_emb, w_gcn0)


def _mm_body(h0_ref, h1_ref, w_ref, o0_ref, o1_ref):
    h = jnp.concatenate([h0_ref[...], h1_ref[...]], axis=1)
    m = jnp.dot(h, w_ref[...], preferred_element_type=jnp.float32)
    o0_ref[...] = m[:, :32]
    o1_ref[...] = m[:, 32:]


def _tc_mm(h0, h1, w):
    return pl.pallas_call(
        _mm_body,
        grid=(GRID,),
        in_specs=[pl.BlockSpec((R, 32), lambda i: (i, 0)),
                  pl.BlockSpec((R, 32), lambda i: (i, 0)),
                  pl.BlockSpec((64, 64), lambda i: (0, 0))],
        out_specs=(pl.BlockSpec((R, 32), lambda i: (i, 0)),
                   pl.BlockSpec((R, 32), lambda i: (i, 0))),
        out_shape=(_H32, _H32),
    )(h0, h1, w)


def _head_body(g0_ref, g1_ref, wp1_ref, wp2_ref, b_ref, o_ref):
    g = jnp.concatenate([g0_ref[...], g1_ref[...]], axis=1)
    z = jnp.maximum(jnp.dot(g, wp1_ref[...],
                            preferred_element_type=jnp.float32), 0.0)
    o_ref[...] = (jnp.dot(z, wp2_ref[...],
                          preferred_element_type=jnp.float32) + b_ref[...])


def _tc_head(g0, g1, wp1, wp2, b2):
    return pl.pallas_call(
        _head_body,
        grid=(1,),
        in_specs=[pl.BlockSpec((B, 32), lambda i: (0, 0)),
                  pl.BlockSpec((B, 32), lambda i: (0, 0)),
                  pl.BlockSpec((64, 32), lambda i: (0, 0)),
                  pl.BlockSpec((32, 1), lambda i: (0, 0)),
                  pl.BlockSpec((1, 1), lambda i: (0, 0))],
        out_specs=pl.BlockSpec((B, 1), lambda i: (0, 0)),
        out_shape=jax.ShapeDtypeStruct((B, 1), jnp.float32),
    )(g0, g1, wp1, wp2, b2)


def kernel(x, edge_index, graph_ids, W_emb, W_gcn0, W_gcn1, W_gcn2,
           W_p1, W_p2, b_p2):
    e3 = edge_index.reshape(2, E // K, K)
    zeros32 = jnp.zeros((WB, 32), jnp.float32)
    gid3 = graph_ids.reshape(NGRP, 2, K)
    b2 = b_p2.reshape(1, 1)

    m0_0, m0_1 = _tc_emb(x, W_emb, W_gcn0)
    h1_0, h1_1 = _sc_layer1(m0_0, m0_1, e3, zeros32)
    m1_0, m1_1 = _tc_mm(h1_0, h1_1, W_gcn1)
    h2_0, h2_1 = _sc_layer2(m1_0, m1_1, e3, zeros32, h1_0, h1_1)
    m2_0, m2_1 = _tc_mm(h2_0, h2_1, W_gcn2)
    g0, g1 = _sc_layer3(m2_0, m2_1, e3, zeros32, h2_0, h2_1, gid3)
    return _tc_head(g0, g1, W_p1, W_p2, b2)


# speedup vs baseline: 1.0145x; 1.0145x over previous
import jax
import jax.numpy as jnp
from jax import lax
from jax.experimental import pallas as pl
from jax.experimental.pallas import tpu as pltpu
from jax.experimental.pallas import tpu_sc as plsc

N = 50000
E = 800000
B = 128
NC = 2
NS = 16

K = 125
NI = 16
P = 6

WB = 3128
WB_LAST = N - 15 * WB

GR = 250
NGRP = N // GR

R = 5000
GRID = N // R


def _zero_acc(zeros_hbm, acc, s):
    @pl.when(s < NS - 1)
    def _():
        pltpu.sync_copy(zeros_hbm, acc.at[pl.ds(s * WB, WB)])

    @pl.when(s == NS - 1)
    def _():
        pltpu.sync_copy(zeros_hbm.at[pl.ds(0, WB_LAST)],
                        acc.at[pl.ds((NS - 1) * WB, WB_LAST)])


def _edge_loop(m_hbm, e3_hbm, acc, sidx, didx, rows, sems, s):
    n_outer = E // (NS * NI * K)
    base_blk = s * n_outer

    def outer(i, carry):
        r0 = (base_blk + i) * NI
        pltpu.sync_copy(e3_hbm.at[0].at[pl.ds(r0, NI)], sidx)
        pltpu.sync_copy(e3_hbm.at[1].at[pl.ds(r0, NI)], didx)
        cps = [None] * NI
        for j in range(P):
            cps[j] = pltpu.async_copy(m_hbm.at[sidx.at[j]], rows.at[j % P],
                                      sems.at[j % P])
        for j in range(NI):
            cps[j].wait()
            pltpu.sync_copy(rows.at[j % P], acc.at[didx.at[j]], add=True)
            if j + P < NI:
                cps[j + P] = pltpu.async_copy(m_hbm.at[sidx.at[j + P]],
                                              rows.at[j % P], sems.at[j % P])
        return carry

    lax.fori_loop(0, n_outer, outer, 0)


def _finish_group(g, acc, hp, out, gid3, g_acc, mode):
    def scoped(vbuf, vbuf2, gbuf):
        r0 = g * GR
        pltpu.sync_copy(acc.at[pl.ds(r0, GR)], vbuf)
        if mode >= 2:
            pltpu.sync_copy(hp.at[pl.ds(r0, GR)], vbuf2)

        def row(r, carry):
            for off in (0, 16):
                v = vbuf[r, pl.ds(off, 16)]
                if mode >= 2:
                    v = v + vbuf2[r, pl.ds(off, 16)]
                vbuf[r, pl.ds(off, 16)] = jnp.maximum(v, 0.0)
            return carry

        lax.fori_loop(0, GR, row, 0)
        if mode == 3:
            pltpu.sync_copy(gid3.at[g], gbuf)
            for j in range(2):
                pltpu.sync_copy(vbuf.at[pl.ds(j * K, K)],
                                g_acc.at[gbuf.at[j]], add=True)
        else:
            pltpu.sync_copy(vbuf, out.at[pl.ds(r0, GR)])

    pl.run_scoped(scoped,
                  pltpu.VMEM((GR, 32), jnp.float32),
                  pltpu.VMEM((GR, 32), jnp.float32),
                  pltpu.VMEM((2, K), jnp.int32))


def _finish(acc, hp, out, gid3, g_acc, s, mode):
    for k in range(12):
        _finish_group(s + NS * k, acc, hp, out, gid3, g_acc, mode)

    @pl.when(s < NGRP - 12 * NS)
    def _():
        _finish_group(12 * NS + s, acc, hp, out, gid3, g_acc, mode)


def _edges_both_cores(m0, m1, e3, acc, sidx, didx, s, c):
    def scoped(rows, sems):
        @pl.when(c == 0)
        def _():
            _edge_loop(m0, e3, acc, sidx, didx, rows, sems, s)

        @pl.when(c == 1)
        def _():
            _edge_loop(m1, e3, acc, sidx, didx, rows, sems, s)

    pl.run_scoped(scoped,
                  pltpu.VMEM((P, K, 32), jnp.float32),
                  pltpu.SemaphoreType.DMA((P,)))


def _layer1_body(m0, m1, e3, zeros, out0, out1, acc, sidx, didx):
    c = lax.axis_index("c")
    s = lax.axis_index("s")
    _zero_acc(zeros, acc, s)
    plsc.subcore_barrier()
    _edges_both_cores(m0, m1, e3, acc, sidx, didx, s, c)
    plsc.subcore_barrier()

    @pl.when(c == 0)
    def _():
        _finish(acc, None, out0, None, None, s, mode=1)

    @pl.when(c == 1)
    def _():
        _finish(acc, None, out1, None, None, s, mode=1)


def _layer2_body(m0, m1, e3, zeros, hp0, hp1, out0, out1, acc, sidx, didx):
    c = lax.axis_index("c")
    s = lax.axis_index("s")
    _zero_acc(zeros, acc, s)
    plsc.subcore_barrier()
    _edges_both_cores(m0, m1, e3, acc, sidx, didx, s, c)
    plsc.subcore_barrier()

    @pl.when(c == 0)
    def _():
        _finish(acc, hp0, out0, None, None, s, mode=2)

    @pl.when(c == 1)
    def _():
        _finish(acc, hp1, out1, None, None, s, mode=2)


def _layer3_body(m0, m1, e3, zeros, hp0, hp1, gid3, gout0, gout1,
                 acc, g_acc, sidx, didx):
    c = lax.axis_index("c")
    s = lax.axis_index("s")
    _zero_acc(zeros, acc, s)

    @pl.when(s == 0)
    def _():
        pltpu.sync_copy(zeros.at[pl.ds(0, B)], g_acc)

    plsc.subcore_barrier()
    _edges_both_cores(m0, m1, e3, acc, sidx, didx, s, c)
    plsc.subcore_barrier()

    @pl.when(c == 0)
    def _():
        _finish(acc, hp0, None, gid3, g_acc, s, mode=3)

    @pl.when(c == 1)
    def _():
        _finish(acc, hp1, None, gid3, g_acc, s, mode=3)

    plsc.subcore_barrier()

    @pl.when((s == 0) & (c == 0))
    def _():
        pltpu.sync_copy(g_acc, gout0)

    @pl.when((s == 0) & (c == 1))
    def _():
        pltpu.sync_copy(g_acc, gout1)


def _sc_mesh():
    return plsc.VectorSubcoreMesh(core_axis_name="c", subcore_axis_name="s",
                                  num_cores=NC, num_subcores=NS)


_H32 = jax.ShapeDtypeStruct((N, 32), jnp.float32)
_G32 = jax.ShapeDtypeStruct((B, 32), jnp.float32)


def _sc_layer1(m0, m1, e3, zeros):
    f = pl.kernel(
        _layer1_body, out_type=(_H32, _H32), mesh=_sc_mesh(),
        compiler_params=pltpu.CompilerParams(use_tc_tiling_on_sc=False),
        scratch_types=[
            pltpu.VMEM_SHARED((N, 32), jnp.float32),
            pltpu.VMEM((NI, K), jnp.int32),
            pltpu.VMEM((NI, K), jnp.int32),
        ],
    )
    return f(m0, m1, e3, zeros)


def _sc_layer2(m0, m1, e3, zeros, hp0, hp1):
    f = pl.kernel(
        _layer2_body, out_type=(_H32, _H32), mesh=_sc_mesh(),
        compiler_params=pltpu.CompilerParams(use_tc_tiling_on_sc=False),
        scratch_types=[
            pltpu.VMEM_SHARED((N, 32), jnp.float32),
            pltpu.VMEM((NI, K), jnp.int32),
            pltpu.VMEM((NI, K), jnp.int32),
        ],
    )
    return f(m0, m1, e3, zeros, hp0, hp1)


def _sc_layer3(m0, m1, e3, zeros, hp0, hp1, gid3):
    f = pl.kernel(
        _layer3_body, out_type=(_G32, _G32), mesh=_sc_mesh(),
        compiler_params=pltpu.CompilerParams(use_tc_tiling_on_sc=False),
        scratch_types=[
            pltpu.VMEM_SHARED((N, 32), jnp.float32),
            pltpu.VMEM_SHARED((B, 32), jnp.float32),
            pltpu.VMEM((NI, K), jnp.int32),
            pltpu.VMEM((NI, K), jnp.int32),
        ],
    )
    return f(m0, m1, e3, zeros, hp0, hp1, gid3)


def _emb_body(x_ref, we_ref, w0_ref, o0_ref, o1_ref):
    h0 = jnp.dot(x_ref[...], we_ref[...], preferred_element_type=jnp.float32)
    m0 = jnp.dot(h0, w0_ref[...], preferred_element_type=jnp.float32)
    o0_ref[...] = m0[:, :32]
    o1_ref[...] = m0[:, 32:]


def _tc_emb(x, w_emb, w_gcn0):
    return pl.pallas_call(
        _emb_body,
        grid=(GRID,),
        in_specs=[pl.BlockSpec((R, 27), lambda i: (i, 0)),
                  pl.BlockSpec((27, 8), lambda i: (0, 0)),
                  pl.BlockSpec((8, 64), lambda i: (0, 0))],
        out_specs=(pl.BlockSpec((R, 32), lambda i: (i, 0)),
                   pl.BlockSpec((R, 32), lambda i: (i, 0))),
        out_shape=(_H32, _H32),
    )(x, w_emb, w_gcn0)


def _mm_body(h0_ref, h1_ref, w_ref, o0_ref, o1_ref):
    h = jnp.concatenate([h0_ref[...], h1_ref[...]], axis=1)
    m = jnp.dot(h, w_ref[...], preferred_element_type=jnp.float32)
    o0_ref[...] = m[:, :32]
    o1_ref[...] = m[:, 32:]


def _tc_mm(h0, h1, w):
    return pl.pallas_call(
        _mm_body,
        grid=(GRID,),
        in_specs=[pl.BlockSpec((R, 32), lambda i: (i, 0)),
                  pl.BlockSpec((R, 32), lambda i: (i, 0)),
                  pl.BlockSpec((64, 64), lambda i: (0, 0))],
        out_specs=(pl.BlockSpec((R, 32), lambda i: (i, 0)),
                   pl.BlockSpec((R, 32), lambda i: (i, 0))),
        out_shape=(_H32, _H32),
    )(h0, h1, w)


def _head_body(g0_ref, g1_ref, wp1_ref, wp2_ref, b_ref, o_ref):
    g = jnp.concatenate([g0_ref[...], g1_ref[...]], axis=1)
    z = jnp.maximum(jnp.dot(g, wp1_ref[...],
                            preferred_element_type=jnp.float32), 0.0)
    o_ref[...] = (jnp.dot(z, wp2_ref[...],
                          preferred_element_type=jnp.float32) + b_ref[...])


def _tc_head(g0, g1, wp1, wp2, b2):
    return pl.pallas_call(
        _head_body,
        grid=(1,),
        in_specs=[pl.BlockSpec((B, 32), lambda i: (0, 0)),
                  pl.BlockSpec((B, 32), lambda i: (0, 0)),
                  pl.BlockSpec((64, 32), lambda i: (0, 0)),
                  pl.BlockSpec((32, 1), lambda i: (0, 0)),
                  pl.BlockSpec((1, 1), lambda i: (0, 0))],
        out_specs=pl.BlockSpec((B, 1), lambda i: (0, 0)),
        out_shape=jax.ShapeDtypeStruct((B, 1), jnp.float32),
    )(g0, g1, wp1, wp2, b2)


def kernel(x, edge_index, graph_ids, W_emb, W_gcn0, W_gcn1, W_gcn2,
           W_p1, W_p2, b_p2):
    e3 = edge_index.reshape(2, E // K, K)
    zeros32 = jnp.zeros((WB, 32), jnp.float32)
    gid3 = graph_ids.reshape(NGRP, 2, K)
    b2 = b_p2.reshape(1, 1)

    m0_0, m0_1 = _tc_emb(x, W_emb, W_gcn0)
    h1_0, h1_1 = _sc_layer1(m0_0, m0_1, e3, zeros32)
    m1_0, m1_1 = _tc_mm(h1_0, h1_1, W_gcn1)
    h2_0, h2_1 = _sc_layer2(m1_0, m1_1, e3, zeros32, h1_0, h1_1)
    m2_0, m2_1 = _tc_mm(h2_0, h2_1, W_gcn2)
    g0, g1 = _sc_layer3(m2_0, m2_1, e3, zeros32, h2_0, h2_1, gid3)
    return _tc_head(g0, g1, W_p1, W_p2, b2)
